# baseline (speedup 1.0000x reference)
import functools

import numpy as np
import jax
import jax.numpy as jnp
from jax import lax
from jax.experimental import pallas as pl
from jax.experimental.pallas import tpu as pltpu
from jax.experimental.pallas import tpu_sc as plsc

N_PIX = 256
NV = 64
PIXSCALE = 1.0
VEL0 = 0.0
DV = 10.0

FOV_HALF = 0.5 * (N_PIX - 1) * PIXSCALE

PLANE = N_PIX * N_PIX
NLIVE = (NV - 1) * PLANE
W_MAIN = 31 * PLANE
LEFT_LO = 62 * PLANE
N_TEC = 16
STRIPE = W_MAIN // N_TEC
ZW = STRIPE // 8

P_CHUNK = 256
N_CALL = (4 * P_CHUNK) // 128


def _floor_f32(x):
    t = x.astype(jnp.int32)
    tf = t.astype(jnp.float32)
    return jnp.where(x < tf, t - 1, t)


def _splat_pairs(ra, dec, vel, flx, lo, size):
    xs = (ra + FOV_HALF) / PIXSCALE
    ys = (dec + FOV_HALF) / PIXSCALE
    vs = (vel - VEL0) / DV
    ix0 = _floor_f32(xs)
    iy0 = _floor_f32(ys)
    iv0 = _floor_f32(vs)
    fx = xs - ix0.astype(jnp.float32)
    fy = ys - iy0.astype(jnp.float32)
    valid = (
        (ix0 >= 0) & (ix0 < N_PIX - 1)
        & (iy0 >= 0) & (iy0 < N_PIX - 1)
        & (iv0 >= 0) & (iv0 < NV - 1)
    )
    ix0c = jnp.clip(ix0, 0, N_PIX - 2)
    iy0c = jnp.clip(iy0, 0, N_PIX - 2)
    iv0c = jnp.clip(iv0, 0, NV - 2)
    wx0 = 1.0 - fx
    wy0 = 1.0 - fy
    a0 = iv0c * PLANE + iy0c * N_PIX
    a1 = a0 + N_PIX
    own0 = valid & (a0 >= lo) & (a0 < lo + size)
    own1 = valid & (a1 >= lo) & (a1 < lo + size)
    a0c = jnp.clip(a0 - lo, 0, size - N_PIX)
    a1c = jnp.clip(a1 - lo, 0, size - N_PIX)
    i00 = a0c + ix0c
    i01 = a1c + ix0c
    zero = jnp.float32(0.0)
    v00 = jnp.where(own0, flx * (wx0 * wy0), zero)
    v01 = jnp.where(own1, flx * (wx0 * fy), zero)
    v10 = jnp.where(own0, flx * (fx * wy0), zero)
    v11 = jnp.where(own1, flx * (fx * fy), zero)
    return (i00, i01, i00 + 1, i01 + 1), (v00, v01, v10, v11)


def _sc_body(ra_h, dec_h, vel_h, flx_h, zeros_h, outm_h, outl_h,
             ra_b, dec_b, vel_b, flx_b, idx_b, val_b, spmem,
             sem_in, sem_sc, ppt):
    c = lax.axis_index("c")
    s = lax.axis_index("s")

    hbufs = (ra_h, dec_h, vel_h, flx_h)
    vbufs = (ra_b, dec_b, vel_b, flx_b)

    def fire_loads(base):
        for h, b in zip(hbufs, vbufs):
            pltpu.async_copy(h.at[pl.ds(base, P_CHUNK)], b, sem_in)


    def splat_phase(pt_base, n_chunks, lo, size):
        fire_loads(pt_base)

        def chunk_body(k, carry):
            base = pt_base + k * P_CHUNK
            for h, b in zip(hbufs, vbufs):
                pltpu.make_async_copy(
                    h.at[pl.ds(base, P_CHUNK)], b, sem_in
                ).wait()
            for cc in range(N_CALL):
                for u in range(2):
                    g = cc * 2 + u
                    sl = pl.ds(g * 16, 16)
                    idxs, vals = _splat_pairs(
                        ra_b[sl], dec_b[sl], vel_b[sl], flx_b[sl], lo, size)
                    for j in range(4):
                        off = u * 64 + j * 16
                        idx_b[cc, pl.ds(off, 16)] = idxs[j]
                        val_b[cc, pl.ds(off, 16)] = vals[j]

            @pl.when(k < n_chunks - 1)
            def _():
                fire_loads(pt_base + (k + 1) * P_CHUNK)

            descs = [
                pltpu.async_copy(val_b.at[cc], spmem.at[idx_b.at[cc]], sem_sc,
                                 add=True)
                for cc in range(N_CALL)
            ]
            for d in descs:
                d.wait()
            return carry

        lax.fori_loop(0, n_chunks, chunk_body, 0)

    with jax.named_scope("sc_zero"):
        for k in range(STRIPE // ZW):
            pltpu.sync_copy(zeros_h, spmem.at[pl.ds(s * STRIPE + k * ZW, ZW)])
        plsc.subcore_barrier()

    with jax.named_scope("sc_main_splat"):
        splat_phase(s * ppt, ppt // P_CHUNK, c * W_MAIN, W_MAIN)
        plsc.subcore_barrier()

    with jax.named_scope("sc_main_drain"):
        pltpu.sync_copy(spmem.at[pl.ds(s * STRIPE, STRIPE)],
                        outm_h.at[pl.ds(c * W_MAIN + s * STRIPE, STRIPE)])
        plsc.subcore_barrier()

    with jax.named_scope("sc_left_zero"):
        pltpu.sync_copy(zeros_h.at[pl.ds(0, PLANE // N_TEC)],
                        spmem.at[pl.ds(s * (PLANE // N_TEC), PLANE // N_TEC)])
        plsc.subcore_barrier()
    ppt_l = ppt // 2
    with jax.named_scope("sc_left_splat"):
        splat_phase((c * N_TEC + s) * ppt_l, ppt_l // P_CHUNK, LEFT_LO, PLANE)
        plsc.subcore_barrier()
    with jax.named_scope("sc_left_drain"):
        pltpu.sync_copy(spmem.at[pl.ds(s * (PLANE // N_TEC), PLANE // N_TEC)],
                        outl_h.at[pl.ds(c * PLANE + s * (PLANE // N_TEC),
                                        PLANE // N_TEC)])


_SELX = np.zeros((256, 128), np.float32)
_SELY = np.zeros((256, 128), np.float32)
for _j in range(128):
    _SELX[2 * _j, _j] = 1.0
    _SELY[2 * _j + 1, _j] = 1.0


def _deint_body(pos_ref, selx_ref, sely_ref, ra_ref, dec_ref):
    blk = pos_ref[...]
    ra2 = jnp.dot(blk, selx_ref[...], precision=lax.Precision.HIGHEST,
                  preferred_element_type=jnp.float32)
    dec2 = jnp.dot(blk, sely_ref[...], precision=lax.Precision.HIGHEST,
                   preferred_element_type=jnp.float32)
    ra_ref[...] = ra2.reshape(-1)
    dec_ref[...] = dec2.reshape(-1)


def _deinterleave(pos2d, m_pad):
    rows_pad = m_pad // 128
    rows = pos2d.shape[0]
    if rows_pad > rows:
        pos2d = jnp.concatenate(
            [pos2d, jnp.zeros((rows_pad - rows, 256), jnp.float32)])
    rb = rows_pad // 8
    return pl.pallas_call(
        _deint_body,
        grid=(8,),
        in_specs=[
            pl.BlockSpec((rb, 256), lambda i: (i, 0)),
            pl.BlockSpec((256, 128), lambda i: (0, 0)),
            pl.BlockSpec((256, 128), lambda i: (0, 0)),
        ],
        out_specs=[
            pl.BlockSpec((rb * 128,), lambda i: (i,)),
            pl.BlockSpec((rb * 128,), lambda i: (i,)),
        ],
        out_shape=[
            jax.ShapeDtypeStruct((m_pad,), jnp.float32),
            jax.ShapeDtypeStruct((m_pad,), jnp.float32),
        ],
    )(pos2d, _SELX, _SELY)


def _splat_cube(pos2d, vel, flx):
    m = vel.shape[0]
    ppt = -(-m // (N_TEC * 2 * P_CHUNK)) * (2 * P_CHUNK)
    m_pad = N_TEC * ppt
    pad = m_pad - m
    ra, dec = _deinterleave(pos2d, m_pad)
    if pad:
        vel = jnp.concatenate([vel, jnp.full((pad,), -100.0, jnp.float32)])
        flx = jnp.concatenate([flx, jnp.zeros((pad,), jnp.float32)])
    zeros_h = jnp.zeros((ZW,), jnp.float32)
    ra, dec, vel, flx, zeros_h = jax.lax.optimization_barrier(
        (ra, dec, vel, flx, zeros_h))

    mesh = plsc.VectorSubcoreMesh(core_axis_name="c", subcore_axis_name="s")
    body = functools.partial(_sc_body, ppt=ppt)
    return pl.kernel(
        body,
        out_type=(
            jax.ShapeDtypeStruct((62 * PLANE,), jnp.float32),
            jax.ShapeDtypeStruct((2 * PLANE,), jnp.float32),
        ),
        mesh=mesh,
        scratch_types=[
            pltpu.VMEM((P_CHUNK,), jnp.float32),
            pltpu.VMEM((P_CHUNK,), jnp.float32),
            pltpu.VMEM((P_CHUNK,), jnp.float32),
            pltpu.VMEM((P_CHUNK,), jnp.float32),
            pltpu.VMEM((N_CALL, 128), jnp.int32),
            pltpu.VMEM((N_CALL, 128), jnp.float32),
            pltpu.VMEM_SHARED((W_MAIN,), jnp.float32),
            pltpu.SemaphoreType.DMA,
            pltpu.SemaphoreType.DMA,
        ],
    )(ra, dec, vel, flx, zeros_h)


def _conv_body(flat_ref, parts_ref, av_ref, ah_ref, out_ref):
    i = pl.program_id(0)
    flat = flat_ref[...]
    p0 = flat[:PLANE].reshape(N_PIX, N_PIX)
    p1 = flat[PLANE:].reshape(N_PIX, N_PIX)
    last = i == NV // 2 - 1
    a = jnp.where(last, parts_ref[0] + parts_ref[1], p0)
    av = av_ref[...]
    ah = ah_ref[...]
    ta = jnp.dot(av, a, preferred_element_type=jnp.float32)
    out_ref[0] = jnp.dot(ta, ah, preferred_element_type=jnp.float32)
    tb = jnp.dot(av, p1, preferred_element_type=jnp.float32)
    res_b = jnp.dot(tb, ah, preferred_element_type=jnp.float32)
    out_ref[1] = jnp.where(last, jnp.float32(0.0), res_b)


def _blur(cube62, parts, kernel2d):
    k2d = kernel2d[0, 0]
    ksz = k2d.shape[0]
    half = ksz // 2
    c = jnp.sqrt(k2d[half, half])
    gcol = k2d[:, half] / c
    grow = k2d[half, :] / c
    av = jnp.zeros((N_PIX, N_PIX), jnp.float32)
    ah = jnp.zeros((N_PIX, N_PIX), jnp.float32)
    for t in range(ksz):
        av = av + gcol[t] * np.eye(N_PIX, k=t - half, dtype=np.float32)
        ah = ah + grow[t] * np.eye(N_PIX, k=-(t - half), dtype=np.float32)
    return pl.pallas_call(
        _conv_body,
        grid=(NV // 2,),
        in_specs=[
            pl.BlockSpec((2 * PLANE,),
                         lambda i: (jnp.minimum(i, NV // 2 - 2),)),
            pl.BlockSpec((2, N_PIX, N_PIX), lambda i: (0, 0, 0)),
            pl.BlockSpec((N_PIX, N_PIX), lambda i: (0, 0)),
            pl.BlockSpec((N_PIX, N_PIX), lambda i: (0, 0)),
        ],
        out_specs=pl.BlockSpec((2, N_PIX, N_PIX), lambda i: (i, 0, 0)),
        out_shape=jax.ShapeDtypeStruct((NV, N_PIX, N_PIX), jnp.float32),
    )(cube62, parts, av, ah)


def kernel(pos_img, vel_chan, flux, kernel2d):
    vel = vel_chan.reshape(-1)
    flx = flux.reshape(-1)
    cube_main, cube_left = _splat_cube(pos_img.reshape(-1, 256), vel, flx)
    parts = cube_left.reshape(2, N_PIX, N_PIX)
    return _blur(cube_main, parts, kernel2d)


# speedup vs baseline: 4.2366x; 4.2366x over previous
import functools

import numpy as np
import jax
import jax.numpy as jnp
from jax import lax
from jax.experimental import pallas as pl
from jax.experimental.pallas import tpu as pltpu
from jax.experimental.pallas import tpu_sc as plsc

N_PIX = 256
NV = 64
PIXSCALE = 1.0
VEL0 = 0.0
DV = 10.0

FOV_HALF = 0.5 * (N_PIX - 1) * PIXSCALE

PLANE = N_PIX * N_PIX
NLIVE = (NV - 1) * PLANE
W_MAIN = 31 * PLANE
LEFT_LO = 62 * PLANE
N_TEC = 16
STRIPE = W_MAIN // N_TEC
ZW = STRIPE // 8

P_CHUNK = 256
N_CALL = (4 * P_CHUNK) // 128


def _floor_f32(x):
    t = x.astype(jnp.int32)
    tf = t.astype(jnp.float32)
    return jnp.where(x < tf, t - 1, t)


def _splat_pairs(ra, dec, vel, flx, lo, size):
    xs = (ra + FOV_HALF) / PIXSCALE
    ys = (dec + FOV_HALF) / PIXSCALE
    vs = (vel - VEL0) / DV
    ix0 = _floor_f32(xs)
    iy0 = _floor_f32(ys)
    iv0 = _floor_f32(vs)
    fx = xs - ix0.astype(jnp.float32)
    fy = ys - iy0.astype(jnp.float32)
    valid = (
        (ix0 >= 0) & (ix0 < N_PIX - 1)
        & (iy0 >= 0) & (iy0 < N_PIX - 1)
        & (iv0 >= 0) & (iv0 < NV - 1)
    )
    ix0c = jnp.clip(ix0, 0, N_PIX - 2)
    iy0c = jnp.clip(iy0, 0, N_PIX - 2)
    iv0c = jnp.clip(iv0, 0, NV - 2)
    wx0 = 1.0 - fx
    wy0 = 1.0 - fy
    a0 = iv0c * PLANE + iy0c * N_PIX
    a1 = a0 + N_PIX
    own0 = valid & (a0 >= lo) & (a0 < lo + size)
    own1 = valid & (a1 >= lo) & (a1 < lo + size)
    a0c = jnp.clip(a0 - lo, 0, size - N_PIX)
    a1c = jnp.clip(a1 - lo, 0, size - N_PIX)
    i00 = a0c + ix0c
    i01 = a1c + ix0c
    zero = jnp.float32(0.0)
    v00 = jnp.where(own0, flx * (wx0 * wy0), zero)
    v01 = jnp.where(own1, flx * (wx0 * fy), zero)
    v10 = jnp.where(own0, flx * (fx * wy0), zero)
    v11 = jnp.where(own1, flx * (fx * fy), zero)
    return (i00, i01, i00 + 1, i01 + 1), (v00, v01, v10, v11)


def _sc_body(ra_h, dec_h, vel_h, flx_h, zeros_h, outm_h, outl_h,
             ra_b, dec_b, vel_b, flx_b, idx_b, val_b, spmem,
             sem_in, sem_sc, ppt):
    c = lax.axis_index("c")
    s = lax.axis_index("s")

    hbufs = (ra_h, dec_h, vel_h, flx_h)
    vbufs = (ra_b, dec_b, vel_b, flx_b)

    def fire_loads(base):
        for h, b in zip(hbufs, vbufs):
            pltpu.async_copy(h.at[pl.ds(base, P_CHUNK)], b, sem_in)


    def splat_phase(pt_base, n_chunks, lo, size):
        fire_loads(pt_base)

        def chunk_body(k, carry):
            base = pt_base + k * P_CHUNK
            for h, b in zip(hbufs, vbufs):
                pltpu.make_async_copy(
                    h.at[pl.ds(base, P_CHUNK)], b, sem_in
                ).wait()
            for cc in range(N_CALL):
                for u in range(2):
                    g = cc * 2 + u
                    sl = pl.ds(g * 16, 16)
                    idxs, vals = _splat_pairs(
                        ra_b[sl], dec_b[sl], vel_b[sl], flx_b[sl], lo, size)
                    for j in range(4):
                        off = u * 64 + j * 16
                        idx_b[cc, pl.ds(off, 16)] = idxs[j]
                        val_b[cc, pl.ds(off, 16)] = vals[j]

            @pl.when(k < n_chunks - 1)
            def _():
                fire_loads(pt_base + (k + 1) * P_CHUNK)

            descs = [
                pltpu.async_copy(val_b.at[cc], spmem.at[idx_b.at[cc]], sem_sc,
                                 add=True)
                for cc in range(N_CALL)
            ]
            for d in descs:
                d.wait()
            return carry

        lax.fori_loop(0, n_chunks, chunk_body, 0)

    with jax.named_scope("sc_zero"):
        for k in range(STRIPE // ZW):
            pltpu.sync_copy(zeros_h, spmem.at[pl.ds(s * STRIPE + k * ZW, ZW)])
        plsc.subcore_barrier()

    with jax.named_scope("sc_main_splat"):
        splat_phase(s * ppt, ppt // P_CHUNK, c * W_MAIN, W_MAIN)
        plsc.subcore_barrier()

    with jax.named_scope("sc_main_drain"):
        pltpu.sync_copy(spmem.at[pl.ds(s * STRIPE, STRIPE)],
                        outm_h.at[pl.ds(c * W_MAIN + s * STRIPE, STRIPE)])
        plsc.subcore_barrier()

    with jax.named_scope("sc_left_zero"):
        pltpu.sync_copy(zeros_h.at[pl.ds(0, PLANE // N_TEC)],
                        spmem.at[pl.ds(s * (PLANE // N_TEC), PLANE // N_TEC)])
        plsc.subcore_barrier()
    ppt_l = ppt // 2
    with jax.named_scope("sc_left_splat"):
        splat_phase((c * N_TEC + s) * ppt_l, ppt_l // P_CHUNK, LEFT_LO, PLANE)
        plsc.subcore_barrier()
    with jax.named_scope("sc_left_drain"):
        pltpu.sync_copy(spmem.at[pl.ds(s * (PLANE // N_TEC), PLANE // N_TEC)],
                        outl_h.at[pl.ds(c * PLANE + s * (PLANE // N_TEC),
                                        PLANE // N_TEC)])


def _splat_cube(ra, dec, vel, flx):
    m = ra.shape[0]
    ppt = -(-m // (N_TEC * 2 * P_CHUNK)) * (2 * P_CHUNK)
    m_pad = N_TEC * ppt
    pad = m_pad - m
    if pad:
        ra = jnp.concatenate([ra, jnp.zeros((pad,), jnp.float32)])
        dec = jnp.concatenate([dec, jnp.zeros((pad,), jnp.float32)])
        vel = jnp.concatenate([vel, jnp.full((pad,), -100.0, jnp.float32)])
        flx = jnp.concatenate([flx, jnp.zeros((pad,), jnp.float32)])
    zeros_h = jnp.zeros((ZW,), jnp.float32)
    ra, dec, vel, flx, zeros_h = jax.lax.optimization_barrier(
        (ra, dec, vel, flx, zeros_h))

    mesh = plsc.VectorSubcoreMesh(core_axis_name="c", subcore_axis_name="s")
    body = functools.partial(_sc_body, ppt=ppt)
    return pl.kernel(
        body,
        out_type=(
            jax.ShapeDtypeStruct((62 * PLANE,), jnp.float32),
            jax.ShapeDtypeStruct((2 * PLANE,), jnp.float32),
        ),
        mesh=mesh,
        scratch_types=[
            pltpu.VMEM((P_CHUNK,), jnp.float32),
            pltpu.VMEM((P_CHUNK,), jnp.float32),
            pltpu.VMEM((P_CHUNK,), jnp.float32),
            pltpu.VMEM((P_CHUNK,), jnp.float32),
            pltpu.VMEM((N_CALL, 128), jnp.int32),
            pltpu.VMEM((N_CALL, 128), jnp.float32),
            pltpu.VMEM_SHARED((W_MAIN,), jnp.float32),
            pltpu.SemaphoreType.DMA,
            pltpu.SemaphoreType.DMA,
        ],
    )(ra, dec, vel, flx, zeros_h)


def _conv_body(flat_ref, parts_ref, av_ref, ah_ref, out_ref):
    i = pl.program_id(0)
    flat = flat_ref[...]
    p0 = flat[:PLANE].reshape(N_PIX, N_PIX)
    p1 = flat[PLANE:].reshape(N_PIX, N_PIX)
    last = i == NV // 2 - 1
    a = jnp.where(last, parts_ref[0] + parts_ref[1], p0)
    av = av_ref[...]
    ah = ah_ref[...]
    ta = jnp.dot(av, a, preferred_element_type=jnp.float32)
    out_ref[0] = jnp.dot(ta, ah, preferred_element_type=jnp.float32)
    tb = jnp.dot(av, p1, preferred_element_type=jnp.float32)
    res_b = jnp.dot(tb, ah, preferred_element_type=jnp.float32)
    out_ref[1] = jnp.where(last, jnp.float32(0.0), res_b)


def _blur(cube62, parts, kernel2d):
    k2d = kernel2d[0, 0]
    ksz = k2d.shape[0]
    half = ksz // 2
    c = jnp.sqrt(k2d[half, half])
    gcol = k2d[:, half] / c
    grow = k2d[half, :] / c
    av = jnp.zeros((N_PIX, N_PIX), jnp.float32)
    ah = jnp.zeros((N_PIX, N_PIX), jnp.float32)
    for t in range(ksz):
        av = av + gcol[t] * np.eye(N_PIX, k=t - half, dtype=np.float32)
        ah = ah + grow[t] * np.eye(N_PIX, k=-(t - half), dtype=np.float32)
    return pl.pallas_call(
        _conv_body,
        grid=(NV // 2,),
        in_specs=[
            pl.BlockSpec((2 * PLANE,),
                         lambda i: (jnp.minimum(i, NV // 2 - 2),)),
            pl.BlockSpec((2, N_PIX, N_PIX), lambda i: (0, 0, 0)),
            pl.BlockSpec((N_PIX, N_PIX), lambda i: (0, 0)),
            pl.BlockSpec((N_PIX, N_PIX), lambda i: (0, 0)),
        ],
        out_specs=pl.BlockSpec((2, N_PIX, N_PIX), lambda i: (i, 0, 0)),
        out_shape=jax.ShapeDtypeStruct((NV, N_PIX, N_PIX), jnp.float32),
    )(cube62, parts, av, ah)


def kernel(pos_img, vel_chan, flux, kernel2d):
    ra = pos_img[..., 0].reshape(-1)
    dec = pos_img[..., 1].reshape(-1)
    vel = vel_chan.reshape(-1)
    flx = flux.reshape(-1)
    cube_main, cube_left = _splat_cube(ra, dec, vel, flx)
    parts = cube_left.reshape(2, N_PIX, N_PIX)
    return _blur(cube_main, parts, kernel2d)
